# baseline (speedup 1.0000x reference)
import functools
import jax
import jax.numpy as jnp
from jax import lax
from jax.experimental import pallas as pl
from jax.experimental.pallas import tpu as pltpu
from jax.experimental.pallas import tpu_sc as plsc

S = 512
H = 128
R = 16
K = 128
T = 1024


def _sc_factor_body(rel_hbm, grep_hbm, colbuf, rowbuf):
    wid = lax.axis_index("s") * 2 + lax.axis_index("c")
    base = jnp.where(wid == 31, 991, 32 * wid)
    pltpu.sync_copy(rel_hbm.at[pl.ds(base * H, 32 * H)], colbuf)
    lane = lax.iota(jnp.int32, 16)
    for lr in range(32):
        acc = jnp.zeros((16,), jnp.float32)
        for c in range(8):
            acc = acc + colbuf[pl.ds(lr * H + c * 16, 16)]
        for k in (8, 4, 2, 1):
            idx = (lane + k) & 15
            acc = acc + acc.at[idx].get(mode="promise_in_bounds")
        g = 1.0 + (0.1 / H) * acc
        for cc in range(8):
            rowbuf[pl.ds((31 - lr) * H + cc * 16, 16)] = g
    pltpu.sync_copy(rowbuf, grep_hbm.at[pl.ds((992 - base) * H, 32 * H)])


def _sc_factor(rel_flat):
    mesh = plsc.VectorSubcoreMesh(core_axis_name="c", subcore_axis_name="s")
    return pl.kernel(
        _sc_factor_body,
        mesh=mesh,
        out_type=jax.ShapeDtypeStruct((T * H,), jnp.float32),
        scratch_types=[
            pltpu.VMEM((32 * H,), jnp.float32),
            pltpu.VMEM((32 * H,), jnp.float32),
        ],
    )(rel_flat)


def _tca_body(pose_ref, pos_ref, rel_ref, out_ref, emb_ref, grep_ref):
    p = pl.program_id(0)

    @pl.when(p == 0)
    def _init():
        emb_ref[...] = pose_ref[0] + pos_ref[...]
        m = jnp.mean(rel_ref[...], axis=1, keepdims=True)
        g = 1.0 + 0.1 * m
        s = jax.lax.broadcasted_iota(jnp.int32, (T, 1), 0)
        for k in (4, 2, 1):
            g = jnp.where((s % (2 * k)) < k,
                          pltpu.roll(g, T - k, 0), pltpu.roll(g, k, 0))
        for b in range(T // 8):
            blk = jax.lax.slice(g, (8 * b, 0), (8 * b + 8, 1))
            grep_ref[pl.ds(8 * (T // 8 - 1 - b), 8), :] = jnp.broadcast_to(
                blk, (8, H))

    i0 = p * R
    emb = emb_ref[...]
    for r in range(R):
        start = S - (i0 + r)
        out_ref[0, r] = emb * grep_ref[pl.ds(start, S), :]


def _tcb_body(pose_ref, pos_ref, grep_ref, outa_ref, out_ref, emb_ref):
    del outa_ref
    p = pl.program_id(0)

    @pl.when(p == 0)
    def _init():
        emb_ref[...] = pose_ref[0] + pos_ref[...]

    i0 = K + p * R
    emb = emb_ref[...]
    for r in range(R):
        start = S - (i0 + r)
        out_ref[0, r] = emb * grep_ref[pl.ds(start, S), :]


def kernel(pose_features, pos_emb_table, rel_table):
    grep = _sc_factor(rel_table.reshape(-1)).reshape(T, H)

    out_shape = jax.ShapeDtypeStruct((1, S, S, H), jnp.float32)
    outa = pl.pallas_call(
        _tca_body,
        grid=(K // R,),
        in_specs=[
            pl.BlockSpec((1, S, H), lambda p: (0, 0, 0)),
            pl.BlockSpec((S, H), lambda p: (0, 0)),
            pl.BlockSpec((T, H), lambda p: (0, 0)),
        ],
        out_specs=pl.BlockSpec((1, R, S, H), lambda p: (0, p, 0, 0)),
        out_shape=out_shape,
        scratch_shapes=[
            pltpu.VMEM((S, H), jnp.float32),
            pltpu.VMEM((T, H), jnp.float32),
        ],
    )(pose_features, pos_emb_table, rel_table)

    out = pl.pallas_call(
        _tcb_body,
        grid=((S - K) // R,),
        in_specs=[
            pl.BlockSpec((1, S, H), lambda p: (0, 0, 0)),
            pl.BlockSpec((S, H), lambda p: (0, 0)),
            pl.BlockSpec((T, H), lambda p: (0, 0)),
            pl.BlockSpec(memory_space=pl.ANY),
        ],
        out_specs=pl.BlockSpec((1, R, S, H), lambda p: (0, K // R + p, 0, 0)),
        out_shape=out_shape,
        scratch_shapes=[pltpu.VMEM((S, H), jnp.float32)],
        input_output_aliases={3: 0},
    )(pose_features, pos_emb_table, grep, outa)
    return out


# speedup vs baseline: 1.4318x; 1.4318x over previous
import functools
import jax
import jax.numpy as jnp
from jax.experimental import pallas as pl
from jax.experimental.pallas import tpu as pltpu

S = 512
H = 128
R = 16
T = 1024


def _body(pose_ref, pos_ref, rel_ref, out_ref, emb_ref, grep_ref):
    p = pl.program_id(0)

    @pl.when(p == 0)
    def _init():
        emb_ref[...] = pose_ref[0] + pos_ref[...]
        m = jnp.mean(rel_ref[...], axis=1, keepdims=True)
        g = 1.0 + 0.1 * m
        s = jax.lax.broadcasted_iota(jnp.int32, (T, 1), 0)
        for k in (4, 2, 1):
            g = jnp.where((s % (2 * k)) < k,
                          pltpu.roll(g, T - k, 0), pltpu.roll(g, k, 0))
        for b in range(T // 8):
            blk = jax.lax.slice(g, (8 * b, 0), (8 * b + 8, 1))
            grep_ref[pl.ds(8 * (T // 8 - 1 - b), 8), :] = jnp.broadcast_to(
                blk, (8, H))

    i0 = p * R
    emb = emb_ref[...]
    for r in range(R):
        start = S - (i0 + r)
        out_ref[0, r] = emb * grep_ref[pl.ds(start, S), :]


def kernel(pose_features, pos_emb_table, rel_table):
    grid = S // R
    out = pl.pallas_call(
        _body,
        grid=(grid,),
        in_specs=[
            pl.BlockSpec((1, S, H), lambda p: (0, 0, 0)),
            pl.BlockSpec((S, H), lambda p: (0, 0)),
            pl.BlockSpec((T, H), lambda p: (0, 0)),
        ],
        out_specs=pl.BlockSpec((1, R, S, H), lambda p: (0, p, 0, 0)),
        out_shape=jax.ShapeDtypeStruct((1, S, S, H), jnp.float32),
        scratch_shapes=[
            pltpu.VMEM((S, H), jnp.float32),
            pltpu.VMEM((T, H), jnp.float32),
        ],
    )(pose_features, pos_emb_table, rel_table)
    return out
